# baseline (speedup 1.0000x reference)
import functools

import jax
import jax.numpy as jnp
from jax import lax
from jax.experimental import pallas as pl
from jax.experimental.pallas import tpu as pltpu
from jax.experimental.pallas import tpu_sc as plsc

N = 65536
NH = 9
E = N * NH

NC, NS, L = 2, 16, 16
NSTREAM = 8
TPS = NC * NS // NSTREAM
EPT = E // TPS
CH = 24576
NSUB = EPT // CH
UNROLL = 8


def _tc_table_body(lon_ref, lat_ref, clon_ref, slon_ref, cl_ref, sl_ref):
    lon = lon_ref[...]
    lat = lat_ref[...]
    clon_ref[...] = jnp.cos(lon)
    slon_ref[...] = jnp.sin(lon)
    cl_ref[...] = jnp.cos(lat)
    sl_ref[...] = jnp.sin(lat)


def _tc_table(lon, lat):
    shape2d = (N // 128, 128)
    outs = pl.pallas_call(
        _tc_table_body,
        out_shape=[jax.ShapeDtypeStruct(shape2d, jnp.float32)] * 4,
        name="tc_node_table",
    )(lon.reshape(shape2d), lat.reshape(shape2d))
    return [o.reshape(N) for o in outs]


def _sc_gather_body(clon_h, slon_h, cl_h, sl_h, idx_h, rep_h,
                    o0, o1, o2, o3, o4, o5, o6, o7,
                    table_v, idx_v, out_v):
    c = lax.axis_index("c")
    s = lax.axis_index("s")
    wid = s * NC + c
    stream = wid // TPS
    quarter = wid % TPS
    plane = stream % 4
    is_self = stream // 4

    for p, src in enumerate((clon_h, slon_h, cl_h, sl_h)):
        @pl.when(plane == p)
        def _(src=src):
            pltpu.sync_copy(src, table_v)

    outs = (o0, o1, o2, o3, o4, o5, o6, o7)
    for sub in range(NSUB):
        off = quarter * EPT + sub * CH

        @pl.when(is_self == 0)
        def _():
            pltpu.sync_copy(idx_h.at[pl.ds(off, CH)], idx_v)

        @pl.when(is_self == 1)
        def _():
            pltpu.sync_copy(rep_h.at[pl.ds(off, CH)], idx_v)

        def body(i, _):
            base = i * (L * UNROLL)
            for u in range(UNROLL):
                o = base + u * L
                iv = idx_v[pl.ds(o, L)]
                out_v[pl.ds(o, L)] = plsc.load_gather(table_v, [iv])
            return 0

        lax.fori_loop(0, CH // (L * UNROLL), body, 0)

        for k in range(NSTREAM):
            @pl.when(stream == k)
            def _(k=k):
                pltpu.sync_copy(out_v, outs[k].at[pl.ds(off, CH)])


@jax.jit
def _sc_gather(clon, slon, cl, sl, idx, rep):
    mesh = plsc.VectorSubcoreMesh(core_axis_name="c", subcore_axis_name="s",
                                  num_cores=NC, num_subcores=NS)
    f = pl.kernel(
        _sc_gather_body,
        out_type=[jax.ShapeDtypeStruct((E,), jnp.float32)] * 8,
        mesh=mesh,
        compiler_params=pltpu.CompilerParams(needs_layout_passes=False),
        scratch_types=[
            pltpu.VMEM((N,), jnp.float32),
            pltpu.VMEM((CH,), jnp.int32),
            pltpu.VMEM((CH,), jnp.float32),
        ],
        name="sc_nh_gather",
    )
    return f(clon, slon, cl, sl, idx, rep)


def _tc_trig_body(clon2_r, slon2_r, cl2_r, sl2_r,
                  clon1_r, slon1_r, cl1_r, sl1_r, out_r):
    clon2 = clon2_r[...]
    slon2 = slon2_r[...]
    cl2 = cl2_r[...]
    sl2 = sl2_r[...]
    clon1 = clon1_r[...]
    slon1 = slon1_r[...]
    cl1 = cl1_r[...]
    sl1 = sl1_r[...]

    cosd = clon2 * clon1 + slon2 * slon1
    sind = slon2 * clon1 - clon2 * slon1
    x = cl2 * cosd
    y = cl2 * sind
    z = sl2
    xr = cl1 * x + sl1 * z
    zr = -sl1 * x + cl1 * z
    dist = jnp.arctan2(jnp.sqrt(y * y + zr * zr), xr)
    theta = jnp.arctan2(zr, y)

    selfm = ((clon2 == clon1) & (slon2 == slon1)
             & (cl2 == cl1) & (sl2 == sl1))
    dist = jnp.where(selfm, 0.0, dist)
    theta = jnp.where(selfm, 0.0, theta)

    lane = lax.broadcasted_iota(jnp.int32, dist.shape, 1)
    even = (lane % 2) == 0
    idx_lo = lane // 2
    idx_hi = idx_lo + 64
    lo = jnp.where(even, jnp.take_along_axis(dist, idx_lo, axis=1),
                   jnp.take_along_axis(theta, idx_lo, axis=1))
    hi = jnp.where(even, jnp.take_along_axis(dist, idx_hi, axis=1),
                   jnp.take_along_axis(theta, idx_hi, axis=1))
    br = dist.shape[0]
    out_r[...] = jnp.stack([lo, hi], axis=1).reshape(2 * br, 128)


_TC_ROWS = E // 128
_TC_BLOCK = 256


@jax.jit
def _tc_trig(planes):
    shape2d = (_TC_ROWS, 128)
    in_spec = pl.BlockSpec((_TC_BLOCK, 128), lambda i: (i, 0))
    out_spec = pl.BlockSpec((2 * _TC_BLOCK, 128), lambda i: (i, 0))
    return pl.pallas_call(
        _tc_trig_body,
        grid=(_TC_ROWS // _TC_BLOCK,),
        in_specs=[in_spec] * 8,
        out_specs=out_spec,
        out_shape=jax.ShapeDtypeStruct((2 * _TC_ROWS, 128), jnp.float32),
        name="tc_rel_trig",
    )(*(p.reshape(shape2d) for p in planes))


def kernel(coordinates, adjc):
    lon = coordinates[:, 0]
    lat = coordinates[:, 1]
    idx = adjc.reshape(-1)
    rep = jnp.repeat(jnp.arange(N, dtype=jnp.int32), NH)
    clon, slon, cl, sl = _tc_table(lon, lat)
    planes = _sc_gather(clon, slon, cl, sl, idx, rep)
    flat = _tc_trig(planes)
    return flat.reshape(N, NH, 2)


# speedup vs baseline: 4.9643x; 4.9643x over previous
import functools

import jax
import jax.numpy as jnp
from jax import lax
from jax.experimental import pallas as pl
from jax.experimental.pallas import tpu as pltpu
from jax.experimental.pallas import tpu_sc as plsc

N = 65536
NH = 9
E = N * NH

NC, NS, L = 2, 16, 16
NPLANE = 4
NRANGE = NC * NS // NPLANE
NPR = N // NRANGE
CN = 2048
NSUB = NPR // CN


def _tc_table_body(lon_ref, lat_ref, clon_ref, slon_ref, cl_ref, sl_ref):
    lon = lon_ref[...]
    lat = lat_ref[...]
    clon_ref[...] = jnp.cos(lon)
    slon_ref[...] = jnp.sin(lon)
    cl_ref[...] = jnp.cos(lat)
    sl_ref[...] = jnp.sin(lat)


def _tc_table(lon, lat):
    shape2d = (N // 128, 128)
    return pl.pallas_call(
        _tc_table_body,
        out_shape=[jax.ShapeDtypeStruct(shape2d, jnp.float32)] * 4,
        name="tc_node_table",
    )(lon.reshape(shape2d), lat.reshape(shape2d))


def _sc_gather_body(clon_h, slon_h, cl_h, sl_h, adjt_h,
                    o0, o1, o2, o3, table_v, idx_v, out_v):
    c = lax.axis_index("c")
    s = lax.axis_index("s")
    wid = s * NC + c
    plane = wid // NRANGE
    rng = wid % NRANGE

    for p, src in enumerate((clon_h, slon_h, cl_h, sl_h)):
        @pl.when(plane == p)
        def _(src=src):
            pltpu.sync_copy(src, table_v)

    outs = (o0, o1, o2, o3)
    for sub in range(NSUB):
        n0 = rng * NPR + sub * CN
        for j in range(NH):
            pltpu.sync_copy(adjt_h.at[pl.ds(j * N + n0, CN)],
                            idx_v.at[pl.ds(j * CN, CN)])

        def body(i, _):
            for j in range(NH):
                o = j * CN + i * L
                iv = idx_v[pl.ds(o, L)]
                out_v[pl.ds(o, L)] = plsc.load_gather(table_v, [iv])
            return 0

        lax.fori_loop(0, CN // L, body, 0)

        for k in range(NPLANE):
            @pl.when(plane == k)
            def _(k=k):
                for j in range(NH):
                    pltpu.sync_copy(out_v.at[pl.ds(j * CN, CN)],
                                    outs[k].at[pl.ds(j * N + n0, CN)])


@jax.jit
def _sc_gather(clon, slon, cl, sl, adjt):
    mesh = plsc.VectorSubcoreMesh(core_axis_name="c", subcore_axis_name="s",
                                  num_cores=NC, num_subcores=NS)
    f = pl.kernel(
        _sc_gather_body,
        out_type=[jax.ShapeDtypeStruct((NH * N,), jnp.float32)] * 4,
        mesh=mesh,
        compiler_params=pltpu.CompilerParams(needs_layout_passes=False),
        scratch_types=[
            pltpu.VMEM((N,), jnp.float32),
            pltpu.VMEM((NH * CN,), jnp.int32),
            pltpu.VMEM((NH * CN,), jnp.float32),
        ],
        name="sc_nh_gather",
    )
    return f(clon, slon, cl, sl, adjt)


def _tc_trig_body(clon2_r, slon2_r, cl2_r, sl2_r,
                  clon1_r, slon1_r, cl1_r, sl1_r, out_r):
    clon2 = clon2_r[...]
    slon2 = slon2_r[...]
    cl2 = cl2_r[...]
    sl2 = sl2_r[...]
    clon1 = clon1_r[...]
    slon1 = slon1_r[...]
    cl1 = cl1_r[...]
    sl1 = sl1_r[...]

    cosd = clon2 * clon1 + slon2 * slon1
    sind = slon2 * clon1 - clon2 * slon1
    x = cl2 * cosd
    y = cl2 * sind
    z = sl2
    xr = cl1 * x + sl1 * z
    zr = -sl1 * x + cl1 * z
    dist = jnp.arctan2(jnp.sqrt(y * y + zr * zr), xr)
    theta = jnp.arctan2(zr, y)

    selfm = ((clon2 == clon1) & (slon2 == slon1)
             & (cl2 == cl1) & (sl2 == sl1))
    dist = jnp.where(selfm, 0.0, dist)
    theta = jnp.where(selfm, 0.0, theta)

    br = dist.shape[0]
    out_r[...] = jnp.stack([dist, theta], axis=1).reshape(1, 2 * br, 128)


_NB = N // 128
_BNB = 64
_GN = _NB // _BNB


@jax.jit
def _tc_trig(nbr_planes, self_planes):
    nbr_spec = pl.BlockSpec((_BNB, 128), lambda nb, j: (j * _GN + nb, 0))
    self_spec = pl.BlockSpec((_BNB, 128), lambda nb, j: (nb, 0))
    out_spec = pl.BlockSpec((1, 2 * _BNB, 128), lambda nb, j: (j, nb, 0))
    return pl.pallas_call(
        _tc_trig_body,
        grid=(_GN, NH),
        in_specs=[nbr_spec] * 4 + [self_spec] * 4,
        out_specs=out_spec,
        out_shape=jax.ShapeDtypeStruct((NH, 2 * _NB, 128), jnp.float32),
        name="tc_rel_trig",
    )(*(p.reshape(NH * _NB, 128) for p in nbr_planes), *self_planes)


def kernel(coordinates, adjc):
    lon = coordinates[:, 0]
    lat = coordinates[:, 1]
    adjt = jnp.swapaxes(adjc, 0, 1).reshape(-1)
    clon, slon, cl, sl = _tc_table(lon, lat)
    planes1d = [p.reshape(N) for p in (clon, slon, cl, sl)]
    nbr = _sc_gather(*planes1d, adjt)
    out3 = _tc_trig(nbr, (clon, slon, cl, sl))
    return (out3.reshape(NH, _NB, 2, 128)
            .transpose(1, 3, 0, 2)
            .reshape(N, NH, 2))


# speedup vs baseline: 7.5353x; 1.5179x over previous
import functools

import jax
import jax.numpy as jnp
from jax import lax
from jax.experimental import pallas as pl
from jax.experimental.pallas import tpu as pltpu
from jax.experimental.pallas import tpu_sc as plsc

N = 65536
NH = 9
E = N * NH

NC, NS, L = 2, 16, 16
NPLANE = 4
NRANGE = NC * NS // NPLANE
NPR = N // NRANGE
CN = 1024
NSUB = NPR // CN


def _tc_table_body(lon_ref, lat_ref, clon_ref, slon_ref, cl_ref, sl_ref):
    lon = lon_ref[...]
    lat = lat_ref[...]
    clon_ref[...] = jnp.cos(lon)
    slon_ref[...] = jnp.sin(lon)
    cl_ref[...] = jnp.cos(lat)
    sl_ref[...] = jnp.sin(lat)


def _tc_table(lon, lat):
    shape2d = (N // 128, 128)
    return pl.pallas_call(
        _tc_table_body,
        out_shape=[jax.ShapeDtypeStruct(shape2d, jnp.float32)] * 4,
        name="tc_node_table",
    )(lon.reshape(shape2d), lat.reshape(shape2d))


def _sc_gather_body(clon_h, slon_h, cl_h, sl_h, adjt_h, out_h,
                    table_v, idx_v, out_v, sem_i, sem_o):
    c = lax.axis_index("c")
    s = lax.axis_index("s")
    wid = s * NC + c
    plane = wid // NRANGE
    rng = wid % NRANGE

    for p, src in enumerate((clon_h, slon_h, cl_h, sl_h)):
        @pl.when(plane == p)
        def _(src=src):
            pltpu.sync_copy(src, table_v)

    BUF = NH * CN
    obase = plane * (NH * N)

    def issue_idx(sub, buf):
        n0 = rng * NPR + sub * CN
        return [
            pltpu.async_copy(adjt_h.at[pl.ds(j * N + n0, CN)],
                             idx_v.at[pl.ds(buf * BUF + j * CN, CN)], sem_i)
            for j in range(NH)
        ]

    def issue_out(sub, buf):
        n0 = rng * NPR + sub * CN
        return [
            pltpu.async_copy(out_v.at[pl.ds(buf * BUF + j * CN, CN)],
                             out_h.at[pl.ds(obase + j * N + n0, CN)], sem_o)
            for j in range(NH)
        ]

    pend_idx = {0: issue_idx(0, 0)}
    pend_out = {}
    for sub in range(NSUB):
        cur = sub % 2
        if sub + 1 < NSUB:
            pend_idx[sub + 1] = issue_idx(sub + 1, (sub + 1) % 2)
        for cd in pend_idx.pop(sub):
            cd.wait()
        if sub - 2 in pend_out:
            for cd in pend_out.pop(sub - 2):
                cd.wait()

        def body(i, _):
            for j in range(NH):
                o = cur * BUF + j * CN + i * L
                iv = idx_v[pl.ds(o, L)]
                out_v[pl.ds(o, L)] = plsc.load_gather(table_v, [iv])
            return 0

        lax.fori_loop(0, CN // L, body, 0)

        pend_out[sub] = issue_out(sub, cur)

    for sub in sorted(pend_out):
        for cd in pend_out[sub]:
            cd.wait()


@jax.jit
def _sc_gather(clon, slon, cl, sl, adjt):
    mesh = plsc.VectorSubcoreMesh(core_axis_name="c", subcore_axis_name="s",
                                  num_cores=NC, num_subcores=NS)
    f = pl.kernel(
        _sc_gather_body,
        out_type=jax.ShapeDtypeStruct((NPLANE * NH * N,), jnp.float32),
        mesh=mesh,
        compiler_params=pltpu.CompilerParams(needs_layout_passes=False),
        scratch_types=[
            pltpu.VMEM((N,), jnp.float32),
            pltpu.VMEM((2 * NH * CN,), jnp.int32),
            pltpu.VMEM((2 * NH * CN,), jnp.float32),
            pltpu.SemaphoreType.DMA,
            pltpu.SemaphoreType.DMA,
        ],
        name="sc_nh_gather",
    )
    return f(clon, slon, cl, sl, adjt)


def _tc_trig_body(clon2_r, slon2_r, cl2_r, sl2_r,
                  clon1_r, slon1_r, cl1_r, sl1_r, out_r):
    clon2 = clon2_r[...]
    slon2 = slon2_r[...]
    cl2 = cl2_r[...]
    sl2 = sl2_r[...]
    clon1 = clon1_r[...]
    slon1 = slon1_r[...]
    cl1 = cl1_r[...]
    sl1 = sl1_r[...]

    cosd = clon2 * clon1 + slon2 * slon1
    sind = slon2 * clon1 - clon2 * slon1
    x = cl2 * cosd
    y = cl2 * sind
    z = sl2
    xr = cl1 * x + sl1 * z
    zr = -sl1 * x + cl1 * z
    dist = jnp.arctan2(jnp.sqrt(y * y + zr * zr), xr)
    theta = jnp.arctan2(zr, y)

    selfm = ((clon2 == clon1) & (slon2 == slon1)
             & (cl2 == cl1) & (sl2 == sl1))
    dist = jnp.where(selfm, 0.0, dist)
    theta = jnp.where(selfm, 0.0, theta)

    br = dist.shape[0]
    out_r[...] = jnp.stack([dist, theta], axis=1).reshape(1, 2 * br, 128)


_NB = N // 128
_BNB = 128
_GN = _NB // _BNB


@jax.jit
def _tc_trig(nbr, self_planes):
    nbr_specs = [
        pl.BlockSpec((_BNB, 128),
                     lambda nb, j, p=p: (p * (NH * _NB // _BNB) + j * _GN + nb, 0))
        for p in range(NPLANE)
    ]
    self_spec = pl.BlockSpec((_BNB, 128), lambda nb, j: (nb, 0))
    out_spec = pl.BlockSpec((1, 2 * _BNB, 128), lambda nb, j: (j, nb, 0))
    nbr2d = nbr.reshape(NPLANE * NH * _NB, 128)
    return pl.pallas_call(
        _tc_trig_body,
        grid=(_GN, NH),
        in_specs=nbr_specs + [self_spec] * 4,
        out_specs=out_spec,
        out_shape=jax.ShapeDtypeStruct((NH, 2 * _NB, 128), jnp.float32),
        name="tc_rel_trig",
    )(nbr2d, nbr2d, nbr2d, nbr2d, *self_planes)


def kernel(coordinates, adjc):
    lon = coordinates[:, 0]
    lat = coordinates[:, 1]
    adjt = jnp.swapaxes(adjc, 0, 1).reshape(-1)
    clon, slon, cl, sl = _tc_table(lon, lat)
    planes1d = [p.reshape(N) for p in (clon, slon, cl, sl)]
    nbr = _sc_gather(*planes1d, adjt)
    out3 = _tc_trig(nbr, (clon, slon, cl, sl))
    return (out3.reshape(NH, _NB, 2, 128)
            .transpose(1, 3, 0, 2)
            .reshape(N, NH, 2))


# speedup vs baseline: 8.5218x; 1.1309x over previous
import functools

import jax
import jax.numpy as jnp
from jax import lax
from jax.experimental import pallas as pl
from jax.experimental.pallas import tpu as pltpu
from jax.experimental.pallas import tpu_sc as plsc

N = 65536
NH = 9
E = N * NH

NC, NS, L = 2, 16, 16
NPLANE = 4
NRANGE = NC * NS // NPLANE
NPR = N // NRANGE
CN = 1024
NSUB = NPR // CN


def _tc_table_body(lon_ref, lat_ref, clon_ref, slon_ref, cl_ref, sl_ref):
    lon = lon_ref[...]
    lat = lat_ref[...]
    clon_ref[...] = jnp.cos(lon)
    slon_ref[...] = jnp.sin(lon)
    cl_ref[...] = jnp.cos(lat)
    sl_ref[...] = jnp.sin(lat)


def _tc_table(lon, lat):
    shape2d = (N // 128, 128)
    return pl.pallas_call(
        _tc_table_body,
        out_shape=[jax.ShapeDtypeStruct(shape2d, jnp.float32)] * 4,
        name="tc_node_table",
    )(lon.reshape(shape2d), lat.reshape(shape2d))


def _sc_gather_body(clon_h, slon_h, cl_h, sl_h, adjt_h, out_h,
                    table_v, idx_v, out_v, sem_i, sem_o):
    c = lax.axis_index("c")
    s = lax.axis_index("s")
    wid = s * NC + c
    plane = wid // NRANGE
    rng = wid % NRANGE

    for p, src in enumerate((clon_h, slon_h, cl_h, sl_h)):
        @pl.when(plane == p)
        def _(src=src):
            pltpu.sync_copy(src, table_v)

    BUF = NH * CN
    obase = plane * (NH * N)

    def issue_idx(sub, buf):
        n0 = rng * NPR + sub * CN
        return [
            pltpu.async_copy(adjt_h.at[pl.ds(j * N + n0, CN)],
                             idx_v.at[pl.ds(buf * BUF + j * CN, CN)], sem_i)
            for j in range(NH)
        ]

    def issue_out(sub, buf):
        n0 = rng * NPR + sub * CN
        return [
            pltpu.async_copy(out_v.at[pl.ds(buf * BUF + j * CN, CN)],
                             out_h.at[pl.ds(obase + j * N + n0, CN)], sem_o)
            for j in range(NH)
        ]

    pend_idx = {0: issue_idx(0, 0)}
    pend_out = {}
    for sub in range(NSUB):
        cur = sub % 2
        if sub + 1 < NSUB:
            pend_idx[sub + 1] = issue_idx(sub + 1, (sub + 1) % 2)
        for cd in pend_idx.pop(sub):
            cd.wait()
        if sub - 2 in pend_out:
            for cd in pend_out.pop(sub - 2):
                cd.wait()

        def body(i, _):
            for j in range(NH):
                o = cur * BUF + j * CN + i * L
                iv = idx_v[pl.ds(o, L)]
                out_v[pl.ds(o, L)] = plsc.load_gather(table_v, [iv])
            return 0

        lax.fori_loop(0, CN // L, body, 0)

        pend_out[sub] = issue_out(sub, cur)

    for sub in sorted(pend_out):
        for cd in pend_out[sub]:
            cd.wait()


@jax.jit
def _sc_gather(clon, slon, cl, sl, adjt):
    mesh = plsc.VectorSubcoreMesh(core_axis_name="c", subcore_axis_name="s",
                                  num_cores=NC, num_subcores=NS)
    f = pl.kernel(
        _sc_gather_body,
        out_type=jax.ShapeDtypeStruct((NPLANE * NH * N,), jnp.float32),
        mesh=mesh,
        compiler_params=pltpu.CompilerParams(needs_layout_passes=False),
        scratch_types=[
            pltpu.VMEM((N,), jnp.float32),
            pltpu.VMEM((2 * NH * CN,), jnp.int32),
            pltpu.VMEM((2 * NH * CN,), jnp.float32),
            pltpu.SemaphoreType.DMA,
            pltpu.SemaphoreType.DMA,
        ],
        name="sc_nh_gather",
    )
    return f(clon, slon, cl, sl, adjt)


_ATAN_C = (0.9999772284426245, -0.33262305470171505, 0.1935418618951062,
           -0.11643035935544656, 0.0526517002056579, -0.011720885418632587)
_PI = 3.14159265358979
_PI_2 = 1.5707963267948966


def _fast_atan2(y, x):
    ax = jnp.abs(x)
    ay = jnp.abs(y)
    hi = jnp.maximum(ax, ay)
    lo = jnp.minimum(ax, ay)
    t = lo / jnp.maximum(hi, 1e-30)
    t2 = t * t
    p = jnp.float32(_ATAN_C[5])
    for k in (4, 3, 2, 1, 0):
        p = p * t2 + jnp.float32(_ATAN_C[k])
    a = p * t
    a = jnp.where(ay > ax, jnp.float32(_PI_2) - a, a)
    a = jnp.where(x < 0, jnp.float32(_PI) - a, a)
    return jnp.where(y < 0, -a, a)


def _tc_trig_body(clon2_r, slon2_r, cl2_r, sl2_r,
                  clon1_r, slon1_r, cl1_r, sl1_r, out_r):
    clon2 = clon2_r[...]
    slon2 = slon2_r[...]
    cl2 = cl2_r[...]
    sl2 = sl2_r[...]
    clon1 = clon1_r[...]
    slon1 = slon1_r[...]
    cl1 = cl1_r[...]
    sl1 = sl1_r[...]

    cosd = clon2 * clon1 + slon2 * slon1
    sind = slon2 * clon1 - clon2 * slon1
    x = cl2 * cosd
    y = cl2 * sind
    z = sl2
    xr = cl1 * x + sl1 * z
    zr = -sl1 * x + cl1 * z
    dist = _fast_atan2(jnp.sqrt(y * y + zr * zr), xr)
    theta = _fast_atan2(zr, y)

    selfm = ((clon2 == clon1) & (slon2 == slon1)
             & (cl2 == cl1) & (sl2 == sl1))
    dist = jnp.where(selfm, 0.0, dist)
    theta = jnp.where(selfm, 0.0, theta)

    br = dist.shape[0]
    out_r[...] = jnp.stack([dist, theta], axis=1).reshape(1, 2 * br, 128)


_NB = N // 128
_BNB = 256
_GN = _NB // _BNB


@jax.jit
def _tc_trig(nbr, self_planes):
    nbr_specs = [
        pl.BlockSpec((_BNB, 128),
                     lambda nb, j, p=p: (p * (NH * _NB // _BNB) + j * _GN + nb, 0))
        for p in range(NPLANE)
    ]
    self_spec = pl.BlockSpec((_BNB, 128), lambda nb, j: (nb, 0))
    out_spec = pl.BlockSpec((1, 2 * _BNB, 128), lambda nb, j: (j, nb, 0))
    nbr2d = nbr.reshape(NPLANE * NH * _NB, 128)
    return pl.pallas_call(
        _tc_trig_body,
        grid=(_GN, NH),
        in_specs=nbr_specs + [self_spec] * 4,
        out_specs=out_spec,
        out_shape=jax.ShapeDtypeStruct((NH, 2 * _NB, 128), jnp.float32),
        name="tc_rel_trig",
    )(nbr2d, nbr2d, nbr2d, nbr2d, *self_planes)


def kernel(coordinates, adjc):
    lon = coordinates[:, 0]
    lat = coordinates[:, 1]
    adjt = jnp.swapaxes(adjc, 0, 1).reshape(-1)
    clon, slon, cl, sl = _tc_table(lon, lat)
    planes1d = [p.reshape(N) for p in (clon, slon, cl, sl)]
    nbr = _sc_gather(*planes1d, adjt)
    out3 = _tc_trig(nbr, (clon, slon, cl, sl))
    return (out3.reshape(NH, _NB, 2, 128)
            .transpose(1, 3, 0, 2)
            .reshape(N, NH, 2))


# speedup vs baseline: 8.8542x; 1.0390x over previous
import functools

import jax
import jax.numpy as jnp
from jax import lax
from jax.experimental import pallas as pl
from jax.experimental.pallas import tpu as pltpu
from jax.experimental.pallas import tpu_sc as plsc

N = 65536
NH = 9
E = N * NH

NC, NS, L = 2, 16, 16
NPLANE = 4
NRANGE = NC * NS // NPLANE
NPR = N // NRANGE
UNROLL = 8


def _tc_table_body(lon_ref, lat_ref, clon_ref, slon_ref, cl_ref, sl_ref):
    lon = lon_ref[...]
    lat = lat_ref[...]
    clon_ref[...] = jnp.cos(lon)
    slon_ref[...] = jnp.sin(lon)
    cl_ref[...] = jnp.cos(lat)
    sl_ref[...] = jnp.sin(lat)


def _tc_table(lon, lat):
    shape2d = (N // 128, 128)
    return pl.pallas_call(
        _tc_table_body,
        out_shape=[jax.ShapeDtypeStruct(shape2d, jnp.float32)] * 4,
        name="tc_node_table",
    )(lon.reshape(shape2d), lat.reshape(shape2d))


def _sc_gather_body(clon_h, slon_h, cl_h, sl_h, adjt_h, out_h,
                    table_v, idx_v, out_v, sem_i, sem_o):
    c = lax.axis_index("c")
    s = lax.axis_index("s")
    wid = s * NC + c
    plane = wid // NRANGE
    rng = wid % NRANGE

    n0 = rng * NPR
    obase = plane * (NH * N)

    def issue_idx(j, buf):
        return pltpu.async_copy(adjt_h.at[pl.ds(j * N + n0, NPR)],
                                idx_v.at[pl.ds(buf * NPR, NPR)], sem_i)

    def issue_out(j, buf):
        return pltpu.async_copy(out_v.at[pl.ds(buf * NPR, NPR)],
                                out_h.at[pl.ds(obase + j * N + n0, NPR)],
                                sem_o)

    pend_idx = {0: issue_idx(0, 0)}

    for p, src in enumerate((clon_h, slon_h, cl_h, sl_h)):
        @pl.when(plane == p)
        def _(src=src):
            pltpu.sync_copy(src, table_v)

    pend_out = {}
    for j in range(NH):
        cur = j % 2
        if j + 1 < NH:
            pend_idx[j + 1] = issue_idx(j + 1, (j + 1) % 2)
        pend_idx.pop(j).wait()
        if j - 2 in pend_out:
            pend_out.pop(j - 2).wait()

        def body(i, _):
            base = i * (L * UNROLL)
            for u in range(UNROLL):
                o = cur * NPR + base + u * L
                iv = idx_v[pl.ds(o, L)]
                out_v[pl.ds(o, L)] = plsc.load_gather(table_v, [iv])
            return 0

        lax.fori_loop(0, NPR // (L * UNROLL), body, 0)

        pend_out[j] = issue_out(j, cur)

    for j in sorted(pend_out):
        pend_out[j].wait()


@jax.jit
def _sc_gather(clon, slon, cl, sl, adjt):
    mesh = plsc.VectorSubcoreMesh(core_axis_name="c", subcore_axis_name="s",
                                  num_cores=NC, num_subcores=NS)
    f = pl.kernel(
        _sc_gather_body,
        out_type=jax.ShapeDtypeStruct((NPLANE * NH * N,), jnp.float32),
        mesh=mesh,
        compiler_params=pltpu.CompilerParams(needs_layout_passes=False),
        scratch_types=[
            pltpu.VMEM((N,), jnp.float32),
            pltpu.VMEM((2 * NPR,), jnp.int32),
            pltpu.VMEM((2 * NPR,), jnp.float32),
            pltpu.SemaphoreType.DMA,
            pltpu.SemaphoreType.DMA,
        ],
        name="sc_nh_gather",
    )
    return f(clon, slon, cl, sl, adjt)


_ATAN_C = (0.9999772284426245, -0.33262305470171505, 0.1935418618951062,
           -0.11643035935544656, 0.0526517002056579, -0.011720885418632587)
_PI = 3.14159265358979
_PI_2 = 1.5707963267948966


def _fast_atan2(y, x):
    ax = jnp.abs(x)
    ay = jnp.abs(y)
    hi = jnp.maximum(ax, ay)
    lo = jnp.minimum(ax, ay)
    t = lo / jnp.maximum(hi, 1e-30)
    t2 = t * t
    p = jnp.float32(_ATAN_C[5])
    for k in (4, 3, 2, 1, 0):
        p = p * t2 + jnp.float32(_ATAN_C[k])
    a = p * t
    a = jnp.where(ay > ax, jnp.float32(_PI_2) - a, a)
    a = jnp.where(x < 0, jnp.float32(_PI) - a, a)
    return jnp.where(y < 0, -a, a)


def _tc_trig_body(clon2_r, slon2_r, cl2_r, sl2_r,
                  clon1_r, slon1_r, cl1_r, sl1_r, out_r):
    clon2 = clon2_r[...]
    slon2 = slon2_r[...]
    cl2 = cl2_r[...]
    sl2 = sl2_r[...]
    clon1 = clon1_r[...]
    slon1 = slon1_r[...]
    cl1 = cl1_r[...]
    sl1 = sl1_r[...]

    cosd = clon2 * clon1 + slon2 * slon1
    sind = slon2 * clon1 - clon2 * slon1
    x = cl2 * cosd
    y = cl2 * sind
    z = sl2
    xr = cl1 * x + sl1 * z
    zr = -sl1 * x + cl1 * z
    dist = _fast_atan2(jnp.sqrt(y * y + zr * zr), xr)
    theta = _fast_atan2(zr, y)

    selfm = ((clon2 == clon1) & (slon2 == slon1)
             & (cl2 == cl1) & (sl2 == sl1))
    dist = jnp.where(selfm, 0.0, dist)
    theta = jnp.where(selfm, 0.0, theta)

    br = dist.shape[0]
    out_r[...] = jnp.stack([dist, theta], axis=1).reshape(1, 2 * br, 128)


_NB = N // 128
_BNB = 256
_GN = _NB // _BNB


@jax.jit
def _tc_trig(nbr, self_planes):
    nbr_specs = [
        pl.BlockSpec((_BNB, 128),
                     lambda nb, j, p=p: (p * (NH * _NB // _BNB) + j * _GN + nb, 0))
        for p in range(NPLANE)
    ]
    self_spec = pl.BlockSpec((_BNB, 128), lambda nb, j: (nb, 0))
    out_spec = pl.BlockSpec((1, 2 * _BNB, 128), lambda nb, j: (j, nb, 0))
    nbr2d = nbr.reshape(NPLANE * NH * _NB, 128)
    return pl.pallas_call(
        _tc_trig_body,
        grid=(_GN, NH),
        in_specs=nbr_specs + [self_spec] * 4,
        out_specs=out_spec,
        out_shape=jax.ShapeDtypeStruct((NH, 2 * _NB, 128), jnp.float32),
        name="tc_rel_trig",
    )(nbr2d, nbr2d, nbr2d, nbr2d, *self_planes)


def kernel(coordinates, adjc):
    lon = coordinates[:, 0]
    lat = coordinates[:, 1]
    adjt = jnp.swapaxes(adjc, 0, 1).reshape(-1)
    clon, slon, cl, sl = _tc_table(lon, lat)
    nbr = _sc_gather(*(p.reshape(N) for p in (clon, slon, cl, sl)), adjt)
    out3 = _tc_trig(nbr, (clon, slon, cl, sl))
    return (out3.reshape(NH, _NB, 2, 128)
            .transpose(1, 3, 0, 2)
            .reshape(N, NH, 2))


# speedup vs baseline: 8.8684x; 1.0016x over previous
import functools

import jax
import jax.numpy as jnp
from jax import lax
from jax.experimental import pallas as pl
from jax.experimental.pallas import tpu as pltpu
from jax.experimental.pallas import tpu_sc as plsc

N = 65536
NH = 9
E = N * NH

NC, NS, L = 2, 16, 16
NPLANE = 4
NRANGE = NC * NS // NPLANE
NPR = N // NRANGE
UNROLL = 8


def _tc_table_body(lon_ref, lat_ref, clon_ref, slon_ref, cl_ref, sl_ref):
    lon = lon_ref[...]
    lat = lat_ref[...]
    clon_ref[...] = jnp.cos(lon)
    slon_ref[...] = jnp.sin(lon)
    cl_ref[...] = jnp.cos(lat)
    sl_ref[...] = jnp.sin(lat)


def _tc_table(lon, lat):
    shape2d = (N // 128, 128)
    return pl.pallas_call(
        _tc_table_body,
        out_shape=[jax.ShapeDtypeStruct(shape2d, jnp.float32)] * 4,
        name="tc_node_table",
    )(lon.reshape(shape2d), lat.reshape(shape2d))


def _sc_gather_body(clon_h, slon_h, cl_h, sl_h, adjt_h, out_h,
                    table_v, idx_v, out_v, sem_i, sem_o):
    c = lax.axis_index("c")
    s = lax.axis_index("s")
    wid = s * NC + c
    plane = wid // NRANGE
    rng = wid % NRANGE

    n0 = rng * NPR
    obase = plane * (NH * N)

    def issue_idx(j, buf):
        return pltpu.async_copy(adjt_h.at[pl.ds(j * N + n0, NPR)],
                                idx_v.at[pl.ds(buf * NPR, NPR)], sem_i)

    def issue_out(j, buf):
        return pltpu.async_copy(out_v.at[pl.ds(buf * NPR, NPR)],
                                out_h.at[pl.ds(obase + j * N + n0, NPR)],
                                sem_o)

    pend_idx = {0: issue_idx(0, 0)}

    for p, src in enumerate((clon_h, slon_h, cl_h, sl_h)):
        @pl.when(plane == p)
        def _(src=src):
            pltpu.sync_copy(src, table_v)

    pend_out = {}
    for j in range(NH):
        cur = j % 2
        if j + 1 < NH:
            pend_idx[j + 1] = issue_idx(j + 1, (j + 1) % 2)
        pend_idx.pop(j).wait()
        if j - 2 in pend_out:
            pend_out.pop(j - 2).wait()

        def body(i, _):
            base = i * (L * UNROLL)
            for u in range(UNROLL):
                o = cur * NPR + base + u * L
                iv = idx_v[pl.ds(o, L)]
                out_v[pl.ds(o, L)] = plsc.load_gather(table_v, [iv])
            return 0

        lax.fori_loop(0, NPR // (L * UNROLL), body, 0)

        pend_out[j] = issue_out(j, cur)

    for j in sorted(pend_out):
        pend_out[j].wait()


@jax.jit
def _sc_gather(clon, slon, cl, sl, adjt):
    mesh = plsc.VectorSubcoreMesh(core_axis_name="c", subcore_axis_name="s",
                                  num_cores=NC, num_subcores=NS)
    f = pl.kernel(
        _sc_gather_body,
        out_type=jax.ShapeDtypeStruct((NPLANE * NH * N,), jnp.float32),
        mesh=mesh,
        compiler_params=pltpu.CompilerParams(needs_layout_passes=False),
        scratch_types=[
            pltpu.VMEM((N,), jnp.float32),
            pltpu.VMEM((2 * NPR,), jnp.int32),
            pltpu.VMEM((2 * NPR,), jnp.float32),
            pltpu.SemaphoreType.DMA,
            pltpu.SemaphoreType.DMA,
        ],
        name="sc_nh_gather",
    )
    return f(clon, slon, cl, sl, adjt)


_ATAN_C = (0.9999772284426245, -0.33262305470171505, 0.1935418618951062,
           -0.11643035935544656, 0.0526517002056579, -0.011720885418632587)
_PI = 3.14159265358979
_PI_2 = 1.5707963267948966


def _atan_poly(t):
    t2 = t * t
    p = jnp.float32(_ATAN_C[5])
    for k in (4, 3, 2, 1, 0):
        p = p * t2 + jnp.float32(_ATAN_C[k])
    return p * t


def _fast_atan2(y, x):
    ax = jnp.abs(x)
    ay = jnp.abs(y)
    hi = jnp.maximum(ax, ay)
    lo = jnp.minimum(ax, ay)
    a = _atan_poly(lo / jnp.maximum(hi, 1e-30))
    a = jnp.where(ay > ax, jnp.float32(_PI_2) - a, a)
    a = jnp.where(x < 0, jnp.float32(_PI) - a, a)
    return jnp.where(y < 0, -a, a)


def _fast_atan2_pos(y, x):
    ax = jnp.abs(x)
    hi = jnp.maximum(ax, y)
    lo = jnp.minimum(ax, y)
    a = _atan_poly(lo / hi)
    a = jnp.where(y > ax, jnp.float32(_PI_2) - a, a)
    return jnp.where(x < 0, jnp.float32(_PI) - a, a)


def _tc_trig_body(clon2_r, slon2_r, cl2_r, sl2_r,
                  clon1_r, slon1_r, cl1_r, sl1_r, out_r):
    clon2 = clon2_r[...]
    slon2 = slon2_r[...]
    cl2 = cl2_r[...]
    sl2 = sl2_r[...]
    clon1 = clon1_r[...]
    slon1 = slon1_r[...]
    cl1 = cl1_r[...]
    sl1 = sl1_r[...]

    cosd = clon2 * clon1 + slon2 * slon1
    sind = slon2 * clon1 - clon2 * slon1
    x = cl2 * cosd
    y = cl2 * sind
    z = sl2
    xr = cl1 * x + sl1 * z
    zr = -sl1 * x + cl1 * z
    dist = _fast_atan2_pos(jnp.sqrt(y * y + zr * zr), xr)
    theta = _fast_atan2(zr, y)

    selfm = ((clon2 == clon1) & (slon2 == slon1)
             & (cl2 == cl1) & (sl2 == sl1))
    dist = jnp.where(selfm, 0.0, dist)
    theta = jnp.where(selfm, 0.0, theta)

    br = dist.shape[0]
    out_r[...] = jnp.stack([dist, theta], axis=1).reshape(1, 2 * br, 128)


_NB = N // 128
_BNB = 256
_GN = _NB // _BNB


@jax.jit
def _tc_trig(nbr, self_planes):
    nbr_specs = [
        pl.BlockSpec((_BNB, 128),
                     lambda nb, j, p=p: (p * (NH * _NB // _BNB) + j * _GN + nb, 0))
        for p in range(NPLANE)
    ]
    self_spec = pl.BlockSpec((_BNB, 128), lambda nb, j: (nb, 0))
    out_spec = pl.BlockSpec((1, 2 * _BNB, 128), lambda nb, j: (j, nb, 0))
    nbr2d = nbr.reshape(NPLANE * NH * _NB, 128)
    return pl.pallas_call(
        _tc_trig_body,
        grid=(_GN, NH),
        in_specs=nbr_specs + [self_spec] * 4,
        out_specs=out_spec,
        out_shape=jax.ShapeDtypeStruct((NH, 2 * _NB, 128), jnp.float32),
        name="tc_rel_trig",
    )(nbr2d, nbr2d, nbr2d, nbr2d, *self_planes)


def kernel(coordinates, adjc):
    lon = coordinates[:, 0]
    lat = coordinates[:, 1]
    adjt = jnp.swapaxes(adjc, 0, 1).reshape(-1)
    clon, slon, cl, sl = _tc_table(lon, lat)
    nbr = _sc_gather(*(p.reshape(N) for p in (clon, slon, cl, sl)), adjt)
    out3 = _tc_trig(nbr, (clon, slon, cl, sl))
    return (out3.reshape(NH, _NB, 2, 128)
            .transpose(1, 3, 0, 2)
            .reshape(N, NH, 2))
